# Initial kernel scaffold; baseline (speedup 1.0000x reference)
import jax
import jax.numpy as jnp
from jax.experimental import pallas as pl


def kernel(n_id, edge_index, edge_label_index, emb_table, Wl1, bl1, Wr1, Wl2, bl2, Wr2, W1, b1, W2, b2, W3, b3):
    raise NotImplementedError("write your pallas kernel here")


# speedup vs baseline: 2.1242x; 2.1242x over previous
import functools

import jax
import jax.numpy as jnp
from jax import lax
from jax.experimental import pallas as pl
from jax.experimental.pallas import tpu as pltpu
from jax.experimental.pallas import tpu_sc as plsc

N_NODES = 50000
N_EDGES = 800000
N_LABEL = 200000
EMB = 64
HID = 128

NC = 2
NS = 16
NW = NC * NS

BN = 512
NPAD = 50176
TPN = NPAD // NS
EPAD = 819200
EROWS = EPAD // 128
EROWS_PT = EROWS // NS
ENB = 16
ENSUP = EROWS_PT // ENB
LPADH = 212992
LTOT = 2 * LPADH
NIDPAD = 65536


def _mesh():
    return plsc.VectorSubcoreMesh(core_axis_name="c", subcore_axis_name="s")


_SC_PARAMS = pltpu.CompilerParams(use_tc_tiling_on_sc=False)


def _sc_gather(D, idx_rows, group):
    rows_pw = idx_rows // NW
    assert rows_pw % 8 == 0 and 8 % group == 0
    supers = rows_pw // 8

    @functools.partial(
        pl.kernel,
        out_type=jax.ShapeDtypeStruct((idx_rows * 128, D), jnp.float32),
        mesh=_mesh(),
        scratch_types=[
            pltpu.VMEM((8, 128), jnp.int32),
            pltpu.VMEM((group, 128, D), jnp.float32),
            pltpu.SemaphoreType.DMA,
            pltpu.SemaphoreType.DMA,
        ],
        compiler_params=_SC_PARAMS,
    )
    def k(table, idx2d, out, idxv, rowsv, gsem, osem):
        w = lax.axis_index("s") * NC + lax.axis_index("c")
        base = w * rows_pw

        def body(sb, carry):
            r0 = base + sb * 8
            pltpu.sync_copy(idx2d.at[pl.ds(r0, 8)], idxv)
            gh = group // 2
            pend = [None, None]
            for g in range(8 // gh):
                b = g % 2
                if pend[b] is not None:
                    for d in pend[b]:
                        d.wait()
                gds = [
                    pltpu.async_copy(
                        table.at[idxv.at[g * gh + j]],
                        rowsv.at[b * gh + j], gsem,
                    )
                    for j in range(gh)
                ]
                for d in gds:
                    d.wait()
                pend[b] = [
                    pltpu.async_copy(
                        rowsv.at[b * gh + j],
                        out.at[pl.ds((r0 + g * gh + j) * 128, 128)],
                        osem,
                    )
                    for j in range(gh)
                ]
            for p in pend:
                if p is not None:
                    for d in p:
                        d.wait()
            return carry

        lax.fori_loop(0, supers, body, 0)

    return k


def _sc_agg(C, xv_rows, with_deg):
    passes = C // NC
    GR = 4

    out_types = [jax.ShapeDtypeStruct((C, NPAD, 32), jnp.float32)]
    if with_deg:
        out_types.append(jax.ShapeDtypeStruct((2, NPAD, 32), jnp.float32))

    scratch = [
        pltpu.VMEM((ENB, 128), jnp.int32),
        pltpu.VMEM((ENB, 128), jnp.int32),
        pltpu.VMEM((ENB, 128), jnp.int32),
        pltpu.VMEM((GR, 128, 32), jnp.float32),
        pltpu.VMEM_SHARED((NPAD, 32), jnp.float32),
    ]
    if with_deg:
        scratch.append(pltpu.VMEM((128, 32), jnp.float32))
    scratch.append(pltpu.SemaphoreType.DMA)
    scratch.append(pltpu.SemaphoreType.DMA)

    def body(*refs):
        if with_deg:
            (xv, src2d, dst2d, zeros32, ones32, out, deg32,
             srcv, dstv, compv, rowsv, acc, onesv, gsem, ssem) = refs
        else:
            (xv, src2d, dst2d, zeros32, out,
             srcv, dstv, compv, rowsv, acc, gsem, ssem) = refs

        c = lax.axis_index("c")
        s = lax.axis_index("s")
        nbase = s * TPN
        ebase = s * EROWS_PT

        for ci in range(passes):
            chunk = c * passes + ci
            pltpu.sync_copy(
                zeros32.at[pl.ds(nbase, TPN)], acc.at[pl.ds(nbase, TPN)]
            )
            plsc.subcore_barrier()

            def ebody(sb, carry):
                r0 = ebase + sb * ENB
                pltpu.sync_copy(src2d.at[pl.ds(r0, ENB)], srcv)
                pltpu.sync_copy(dst2d.at[pl.ds(r0, ENB)], dstv)
                for j in range(ENB):
                    for i in range(8):
                        v = srcv[j, pl.ds(i * 16, 16)]
                        compv[j, pl.ds(i * 16, 16)] = v * C + chunk
                gh = GR // 2
                pend = [None, None]
                for g in range(ENB // gh):
                    b = g % 2
                    if pend[b] is not None:
                        for d in pend[b]:
                            d.wait()
                    gds = [
                        pltpu.async_copy(
                            xv.at[compv.at[g * gh + j]],
                            rowsv.at[b * gh + j], gsem,
                        )
                        for j in range(gh)
                    ]
                    for d in gds:
                        d.wait()
                    pend[b] = [
                        pltpu.async_copy(
                            rowsv.at[b * gh + j], acc.at[dstv.at[g * gh + j]],
                            ssem, add=True,
                        )
                        for j in range(gh)
                    ]
                for p in pend:
                    if p is not None:
                        for d in p:
                            d.wait()
                return carry

            lax.fori_loop(0, ENSUP, ebody, 0)
            plsc.subcore_barrier()
            pltpu.sync_copy(
                acc.at[pl.ds(nbase, TPN)], out.at[chunk, pl.ds(nbase, TPN)]
            )

        if with_deg:
            pltpu.sync_copy(ones32, onesv)
            pltpu.sync_copy(
                zeros32.at[pl.ds(nbase, TPN)], acc.at[pl.ds(nbase, TPN)]
            )
            plsc.subcore_barrier()
            drows = EROWS // 32
            dbase = c * (EROWS // 2) + s * drows

            def dbody(sb, carry):
                r0 = dbase + sb * 8
                pltpu.sync_copy(dst2d.at[pl.ds(r0, 8)], dstv.at[pl.ds(0, 8)])
                sds = [
                    pltpu.async_copy(
                        onesv, acc.at[dstv.at[j]], ssem, add=True
                    )
                    for j in range(8)
                ]
                for d in sds:
                    d.wait()
                return carry

            lax.fori_loop(0, drows // 8, dbody, 0)
            plsc.subcore_barrier()
            pltpu.sync_copy(
                acc.at[pl.ds(nbase, TPN)], deg32.at[c, pl.ds(nbase, TPN)]
            )

    return functools.partial(
        pl.kernel,
        out_type=tuple(out_types) if with_deg else out_types[0],
        mesh=_mesh(),
        scratch_types=scratch,
        compiler_params=_SC_PARAMS,
    )(body)


def _tc_layer(C, Dx, agg, deg32, x, Wl, bl, Wr):
    D = 32 * C
    grid = NPAD // BN

    def body(agg_ref, deg_ref, x_ref, wl_ref, bl_ref, wr_ref, o_ref):
        degb = deg_ref[...]
        deg = jnp.maximum(degb[0][:, 0:1] + degb[1][:, 0:1], 1.0)
        r = 1.0 / deg
        acc = bl_ref[...] + jnp.dot(
            x_ref[...], wr_ref[...], preferred_element_type=jnp.float32
        )
        for ci in range(C):
            mean = agg_ref[ci] * r
            acc = acc + jnp.dot(
                mean,
                wl_ref[...][ci * 32:(ci + 1) * 32, :],
                preferred_element_type=jnp.float32,
            )
        o_ref[...] = jnp.maximum(acc, 0.0)

    return pl.pallas_call(
        body,
        grid=(grid,),
        in_specs=[
            pl.BlockSpec((C, BN, 32), lambda i: (0, i, 0)),
            pl.BlockSpec((2, BN, 32), lambda i: (0, i, 0)),
            pl.BlockSpec((BN, Dx), lambda i: (i, 0)),
            pl.BlockSpec((D, HID), lambda i: (0, 0)),
            pl.BlockSpec((1, HID), lambda i: (0, 0)),
            pl.BlockSpec((Dx, HID), lambda i: (0, 0)),
        ],
        out_specs=pl.BlockSpec((BN, HID), lambda i: (i, 0)),
        out_shape=jax.ShapeDtypeStruct((NPAD, HID), jnp.float32),
    )(agg, deg32, x, Wl, bl, Wr)


def _tc_head(gath, W1, b1, W2, b2, W3p, b3p):
    grid = 391

    def body(hs_ref, ht_ref, w1_ref, b1_ref, w2_ref, b2_ref, w3_ref,
             b3_ref, o_ref):
        s = hs_ref[...]
        t = ht_ref[...]
        w1 = w1_ref[...]
        e = (
            jnp.dot(s, w1[0:HID], preferred_element_type=jnp.float32)
            + jnp.dot(t, w1[HID:2 * HID], preferred_element_type=jnp.float32)
            + jnp.dot(jnp.abs(s - t), w1[2 * HID:3 * HID],
                      preferred_element_type=jnp.float32)
            + jnp.dot(s * t, w1[3 * HID:4 * HID],
                      preferred_element_type=jnp.float32)
            + b1_ref[...]
        )
        e = jnp.maximum(e, 0.0)
        e = jnp.maximum(
            jnp.dot(e, w2_ref[...], preferred_element_type=jnp.float32)
            + b2_ref[...],
            0.0,
        )
        o_ref[...] = (
            jnp.dot(e, w3_ref[...], preferred_element_type=jnp.float32)
            + b3_ref[...]
        )

    return pl.pallas_call(
        body,
        grid=(grid,),
        in_specs=[
            pl.BlockSpec((BN, HID), lambda i: (i, 0)),
            pl.BlockSpec((BN, HID), lambda i: (i + LPADH // BN, 0)),
            pl.BlockSpec((4 * HID, HID), lambda i: (0, 0)),
            pl.BlockSpec((1, HID), lambda i: (0, 0)),
            pl.BlockSpec((HID, 32), lambda i: (0, 0)),
            pl.BlockSpec((1, 32), lambda i: (0, 0)),
            pl.BlockSpec((32, 8), lambda i: (0, 0)),
            pl.BlockSpec((1, 8), lambda i: (0, 0)),
        ],
        out_specs=pl.BlockSpec((BN, 8), lambda i: (i, 0)),
        out_shape=jax.ShapeDtypeStruct((grid * BN, 8), jnp.float32),
    )(gath, gath, W1, b1, W2, b2, W3p, b3p)


def kernel(n_id, edge_index, edge_label_index, emb_table, Wl1, bl1, Wr1,
           Wl2, bl2, Wr2, W1, b1, W2, b2, W3, b3):
    f32 = jnp.float32
    i32 = jnp.int32

    nid2d = jnp.pad(n_id.astype(i32), (0, NIDPAD - N_NODES)).reshape(-1, 128)
    src2d = jnp.pad(edge_index[0].astype(i32),
                    (0, EPAD - N_EDGES)).reshape(EROWS, 128)
    dst2d = jnp.pad(edge_index[1].astype(i32), (0, EPAD - N_EDGES),
                    constant_values=NPAD - 1).reshape(EROWS, 128)
    lcat = jnp.concatenate([
        jnp.pad(edge_label_index[0].astype(i32), (0, LPADH - N_LABEL)),
        jnp.pad(edge_label_index[1].astype(i32), (0, LPADH - N_LABEL)),
    ]).reshape(-1, 128)
    zeros32 = jnp.zeros((NPAD, 32), f32)
    ones32 = jnp.ones((128, 32), f32)

    bl1r = bl1.reshape(1, HID)
    bl2r = bl2.reshape(1, HID)
    b1r = b1.reshape(1, HID)
    b2r = b2.reshape(1, 32)
    W3p = jnp.pad(W3, ((0, 0), (0, 5)))
    b3p = jnp.pad(b3, (0, 5)).reshape(1, 8)

    x = _sc_gather(EMB, NIDPAD // 128, 8)(emb_table, nid2d)

    xv = x.reshape(-1, 32)
    agg1, deg32 = _sc_agg(2, xv.shape[0], True)(
        xv, src2d, dst2d, zeros32, ones32
    )

    h = _tc_layer(2, EMB, agg1, deg32, x, Wl1, bl1r, Wr1)

    hv = h.reshape(-1, 32)
    agg2 = _sc_agg(4, hv.shape[0], False)(hv, src2d, dst2d, zeros32)

    z = _tc_layer(4, HID, agg2, deg32, h, Wl2, bl2r, Wr2)

    gath = _sc_gather(HID, LTOT // 128, 4)(z, lcat)

    out8 = _tc_head(gath, W1, b1r, W2, b2r, W3p, b3p)
    return out8[:N_LABEL, :3]
